# baseline (speedup 1.0000x reference)
import functools

import jax
import jax.numpy as jnp
from jax.experimental import pallas as pl
from jax.experimental.pallas import tpu as pltpu
from jax.experimental.pallas import tpu_sc as plsc

N = 16384
K = 32
D_IN = 64
H = 128
R = 0.02 * 2.1
R2 = R * R
EPS = 1e-5
SLOPE = 0.01

G = 6
CS = R
NC = 1024
CHUNK = 512
BQ = 128
RBLK = 6
NRUN = 3
RUN_LO = (-43, -7, 29)
RUN_HI = (-29, 7, 43)
W = NRUN * RBLK * 128
BN = 1024
GATHER_WINDOW = 256

_HI = jax.lax.Precision.HIGHEST


def _leaky(v):
    return jnp.where(v >= 0, v, SLOPE * v)


def _cell_body(pq_ref, cid_ref, hist_ref):
    p = pq_ref[...]
    cx = jnp.clip(jnp.floor(p[:, 0:1] / CS), 0, G - 1)
    cy = jnp.clip(jnp.floor(p[:, 1:2] / CS), 0, G - 1)
    cz = jnp.clip(jnp.floor(p[:, 2:3] / CS), 0, G - 1)
    cid = ((p[:, 3:4] * G + cx) * G + cy) * G + cz
    cid_ref[...] = cid
    ci = jax.lax.broadcasted_iota(jnp.int32, (1, NC), 1).astype(jnp.float32)
    onehot = (cid == ci).astype(jnp.float32)

    @pl.when(pl.program_id(0) == 0)
    def _():
        hist_ref[...] = jnp.zeros_like(hist_ref)

    hist_ref[...] += jnp.sum(onehot, axis=0, keepdims=True)


def _cellhist(posq):
    return pl.pallas_call(
        _cell_body,
        grid=(N // CHUNK,),
        in_specs=[pl.BlockSpec((CHUNK, 8), lambda i: (i, 0))],
        out_specs=[
            pl.BlockSpec((CHUNK, 1), lambda i: (i, 0)),
            pl.BlockSpec((1, NC), lambda i: (0, 0)),
        ],
        out_shape=[
            jax.ShapeDtypeStruct((N, 1), jnp.float32),
            jax.ShapeDtypeStruct((1, NC), jnp.float32),
        ],
    )(posq)


def _rank_body(cid_ref, hist_ref, p_ref, base_ref, run_ref):
    ri = jax.lax.broadcasted_iota(jnp.int32, (NC, NC), 0)
    ci = jax.lax.broadcasted_iota(jnp.int32, (NC, NC), 1)

    @pl.when(pl.program_id(0) == 0)
    def _():
        run_ref[...] = jnp.zeros_like(run_ref)
        triu = (ri < ci).astype(jnp.float32)
        base_ref[...] = jax.lax.dot_general(
            hist_ref[...], triu, (((1,), (0,)), ((), ())),
            precision=_HI, preferred_element_type=jnp.float32)

    cid = cid_ref[...]
    lane = jax.lax.broadcasted_iota(jnp.int32, (1, NC), 1).astype(jnp.float32)
    onehot = (cid == lane).astype(jnp.float32)
    rj = jax.lax.broadcasted_iota(jnp.int32, (CHUNK, CHUNK), 0)
    cj = jax.lax.broadcasted_iota(jnp.int32, (CHUNK, CHUNK), 1)
    tril = (cj < rj).astype(jnp.float32)
    excl = jax.lax.dot_general(
        tril, onehot, (((1,), (0,)), ((), ())),
        precision=_HI, preferred_element_type=jnp.float32)
    dest = jnp.sum(onehot * (excl + run_ref[...] + base_ref[...]),
                   axis=1, keepdims=True)
    p_ref[...] = dest.astype(jnp.int32)
    run_ref[...] += jnp.sum(onehot, axis=0, keepdims=True)


def _rankpos(cid, hist):
    return pl.pallas_call(
        _rank_body,
        grid=(N // CHUNK,),
        in_specs=[
            pl.BlockSpec((CHUNK, 1), lambda i: (i, 0)),
            pl.BlockSpec((1, NC), lambda i: (0, 0)),
        ],
        out_specs=[
            pl.BlockSpec((CHUNK, 1), lambda i: (i, 0)),
            pl.BlockSpec((1, NC), lambda i: (0, 0)),
        ],
        out_shape=[
            jax.ShapeDtypeStruct((N, 1), jnp.int32),
            jax.ShapeDtypeStruct((1, NC), jnp.float32),
        ],
        scratch_shapes=[pltpu.VMEM((1, NC), jnp.float32)],
    )(cid, hist)


def _sc_scatter(rows, indices):
    num_idx = indices.shape[1]
    dim = rows.shape[1]
    mesh = plsc.VectorSubcoreMesh(core_axis_name="core",
                                  subcore_axis_name="subcore")

    @functools.partial(
        pl.kernel,
        out_type=jax.ShapeDtypeStruct((num_idx, dim), rows.dtype),
        mesh=mesh,
    )
    def scatter_kernel(x_hbm, i_hbm, o_hbm):
        def body(x_vmem, i_vmem):
            pltpu.sync_copy(x_vmem, o_hbm.at[i_vmem.at[0]])

        pltpu.emit_pipeline(
            body,
            grid=(num_idx // GATHER_WINDOW,),
            in_specs=[
                pl.BlockSpec((GATHER_WINDOW, dim), index_map=lambda i: (i, 0)),
                pl.BlockSpec((1, GATHER_WINDOW), index_map=lambda i: (0, i)),
            ],
            out_specs=[],
            core_axis_name=("core", "subcore"),
            dimension_semantics=(pltpu.PARALLEL,),
        )(x_hbm, i_hbm)

    return scatter_kernel(rows, indices)


def _sc_gather(table, indices):
    num_idx = indices.shape[1]
    dim = table.shape[1]
    mesh = plsc.VectorSubcoreMesh(core_axis_name="core",
                                  subcore_axis_name="subcore")

    @functools.partial(
        pl.kernel,
        out_type=jax.ShapeDtypeStruct((num_idx, dim), table.dtype),
        mesh=mesh,
    )
    def gather_kernel(x_hbm, i_hbm, o_hbm):
        def body(i_vmem, o_vmem):
            pltpu.sync_copy(x_hbm.at[i_vmem.at[0]], o_vmem)

        pltpu.emit_pipeline(
            body,
            grid=(num_idx // GATHER_WINDOW,),
            in_specs=[pl.BlockSpec((1, GATHER_WINDOW), index_map=lambda i: (0, i))],
            out_specs=[pl.BlockSpec((GATHER_WINDOW, dim), index_map=lambda i: (i, 0))],
            core_axis_name=("core", "subcore"),
            dimension_semantics=(pltpu.PARALLEL,),
        )(i_hbm, o_hbm)

    return gather_kernel(table, indices)


def _search_body(meta_ref, q_ref, pb_ref, idx_ref, val_ref):
    pid = pl.program_id(0)
    q = q_ref[...]
    qx = q[:, 0:1]
    qy = q[:, 1:2]
    qz = q[:, 2:3]
    qb = q[:, 3:4]
    cid0 = meta_ref[3, pid]
    cidl = meta_ref[4, pid]
    inf = jnp.float32(jnp.inf)
    parts = []
    giotas = []
    li0 = jax.lax.broadcasted_iota(jnp.int32, (BQ, 128), 1)
    for r in range(NRUN):
        sb = meta_ref[r, pid]
        win = pb_ref[pl.ds(sb, RBLK)]
        lo = (cid0 + RUN_LO[r]).astype(jnp.float32)
        hi = (cidl + RUN_HI[r]).astype(jnp.float32)
        for c in range(RBLK):
            w = win[c]
            dx = qx - w[0:1, :]
            dy = qy - w[1:2, :]
            dz = qz - w[2:3, :]
            d2 = (dx * dx + dy * dy) + dz * dz
            wcid = w[4:5, :]
            m = ((qb == w[3:4, :]) & (d2 <= R2)
                 & (wcid >= lo) & (wcid <= hi))
            parts.append(jnp.where(m, d2, inf))
            giotas.append((sb + c) * 128 + li0)
    d2m0 = jnp.concatenate(parts, axis=1)
    giota = jnp.concatenate(giotas, axis=1)
    kiota = jax.lax.broadcasted_iota(jnp.int32, (BQ, K), 1)

    def step(k, carry):
        d2m, idxa, vala = carry
        v = jnp.min(d2m, axis=1, keepdims=True)
        j = jnp.min(jnp.where(d2m == v, giota, N), axis=1, keepdims=True)
        idxa = jnp.where(kiota == k, j, idxa)
        vala = jnp.where(kiota == k, (v <= R2).astype(jnp.float32), vala)
        d2m = jnp.where(giota == j, inf, d2m)
        return d2m, idxa, vala

    _, idxa, vala = jax.lax.fori_loop(
        0, K, step,
        (d2m0, jnp.zeros((BQ, K), jnp.int32), jnp.zeros((BQ, K), jnp.float32)))
    idx_ref[...] = idxa
    val_ref[...] = vala


def _search(meta, pos_srt, pos_b):
    return pl.pallas_call(
        _search_body,
        grid=(N // BQ,),
        in_specs=[
            pl.BlockSpec(memory_space=pltpu.SMEM),
            pl.BlockSpec((BQ, H), lambda i: (i, 0)),
            pl.BlockSpec((N // 128, 8, 128), lambda i: (0, 0, 0)),
        ],
        out_specs=[
            pl.BlockSpec((BQ, K), lambda i: (i, 0)),
            pl.BlockSpec((BQ, K), lambda i: (i, 0)),
        ],
        out_shape=[
            jax.ShapeDtypeStruct((N, K), jnp.int32),
            jax.ShapeDtypeStruct((N, K), jnp.float32),
        ],
    )(meta, pos_srt, pos_b)


def _pre_body(x_ref, p8_ref, bf_ref, sf_ref, w1a_ref, w1b_ref, b1_ref,
              a_ref, b_ref):
    p8 = p8_ref[...]
    bf = bf_ref[...]
    sfr = sf_ref[...]
    lane8 = jax.lax.broadcasted_iota(jnp.int32, (1, 8), 1)
    onehot = (bf == lane8.astype(jnp.float32)).astype(jnp.float32)
    s = jnp.sum(onehot * sfr, axis=1, keepdims=True)
    recip = 1.0 / s
    psc = jnp.where(lane8 < 3, p8 * recip, p8)
    bmat = jnp.dot(psc, w1b_ref[...], preferred_element_type=jnp.float32)
    amat = (jnp.dot(x_ref[...], w1a_ref[...],
                    preferred_element_type=jnp.float32) + bmat + b1_ref[...])
    a_ref[...] = amat
    b_ref[...] = bmat


def _precompute(x, p8, bf, sf8, w1a, w1b, b1r):
    blk = 2048
    return pl.pallas_call(
        _pre_body,
        grid=(N // blk,),
        in_specs=[
            pl.BlockSpec((blk, D_IN), lambda i: (i, 0)),
            pl.BlockSpec((blk, 8), lambda i: (i, 0)),
            pl.BlockSpec((blk, 1), lambda i: (i, 0)),
            pl.BlockSpec((1, 8), lambda i: (0, 0)),
            pl.BlockSpec((D_IN, H), lambda i: (0, 0)),
            pl.BlockSpec((8, H), lambda i: (0, 0)),
            pl.BlockSpec((1, H), lambda i: (0, 0)),
        ],
        out_specs=[
            pl.BlockSpec((blk, H), lambda i: (i, 0)),
            pl.BlockSpec((blk, H), lambda i: (i, 0)),
        ],
        out_shape=[
            jax.ShapeDtypeStruct((N, H), jnp.float32),
            jax.ShapeDtypeStruct((N, H), jnp.float32),
        ],
    )(x, p8, bf, sf8, w1a, w1b, b1r)


def _mlp_body(e_ref, b_ref, val_ref, w2_ref, b2_ref,
              m_ref, s_ref, s2_ref, cnt_ref):
    e = e_ref[...]
    bi = b_ref[...]
    val = val_ref[...]
    e3 = e.reshape(BN, K, H)
    h1 = _leaky(e3 - bi[:, None, :])
    h1f = h1.reshape(BN * K, H)
    h2 = _leaky(jnp.dot(h1f, w2_ref[...], preferred_element_type=jnp.float32)
                + b2_ref[...])
    h23 = h2.reshape(BN, K, H)
    v3 = val[:, :, None]
    m_ref[...] = jnp.max(jnp.where(v3 > 0, h23, -jnp.inf), axis=1)
    hv = h23 * v3
    s_part = jnp.sum(hv, axis=(0, 1), keepdims=False)
    s2_part = jnp.sum(hv * h23, axis=(0, 1), keepdims=False)
    cnt_part = jnp.sum(val)

    @pl.when(pl.program_id(0) == 0)
    def _():
        s_ref[...] = jnp.zeros_like(s_ref)
        s2_ref[...] = jnp.zeros_like(s2_ref)
        cnt_ref[...] = jnp.zeros_like(cnt_ref)

    s_ref[...] += s_part[None, :]
    s2_ref[...] += s2_part[None, :]
    cnt_ref[...] += jnp.full((1, H), cnt_part, dtype=jnp.float32)


def _mlp(e, bmat, val, w2, b2r):
    return pl.pallas_call(
        _mlp_body,
        grid=(N // BN,),
        in_specs=[
            pl.BlockSpec((BN * K, H), lambda i: (i, 0)),
            pl.BlockSpec((BN, H), lambda i: (i, 0)),
            pl.BlockSpec((BN, K), lambda i: (i, 0)),
            pl.BlockSpec((H, H), lambda i: (0, 0)),
            pl.BlockSpec((1, H), lambda i: (0, 0)),
        ],
        out_specs=[
            pl.BlockSpec((BN, H), lambda i: (i, 0)),
            pl.BlockSpec((1, H), lambda i: (0, 0)),
            pl.BlockSpec((1, H), lambda i: (0, 0)),
            pl.BlockSpec((1, H), lambda i: (0, 0)),
        ],
        out_shape=[
            jax.ShapeDtypeStruct((N, H), jnp.float32),
            jax.ShapeDtypeStruct((1, H), jnp.float32),
            jax.ShapeDtypeStruct((1, H), jnp.float32),
            jax.ShapeDtypeStruct((1, H), jnp.float32),
        ],
    )(e, bmat, val, w2, b2r)


def _fin_body(m_ref, s_ref, s2_ref, cnt_ref, g_ref, be_ref, o_ref):
    cnt = jnp.maximum(cnt_ref[0, 0], 1.0)
    mean = s_ref[...] / cnt
    var = jnp.maximum(s2_ref[...] / cnt - mean * mean, 0.0)
    inv = jax.lax.rsqrt(var + EPS) * g_ref[...]
    o_ref[...] = (m_ref[...] - mean) * inv + be_ref[...]


def _finalize(m, s, s2, cnt, gr, ber):
    blk = 2048
    return pl.pallas_call(
        _fin_body,
        grid=(N // blk,),
        in_specs=[
            pl.BlockSpec((blk, H), lambda i: (i, 0)),
            pl.BlockSpec((1, H), lambda i: (0, 0)),
            pl.BlockSpec((1, H), lambda i: (0, 0)),
            pl.BlockSpec((1, H), lambda i: (0, 0)),
            pl.BlockSpec((1, H), lambda i: (0, 0)),
            pl.BlockSpec((1, H), lambda i: (0, 0)),
        ],
        out_specs=pl.BlockSpec((blk, H), lambda i: (i, 0)),
        out_shape=jax.ShapeDtypeStruct((N, H), jnp.float32),
    )(m, s, s2, cnt, gr, ber)


def kernel(x, pos, batch, reflectance, sf, W1, b1, W2, b2, gamma, beta):
    batch_f = batch.astype(jnp.float32)
    zeros4 = jnp.zeros((N, 4), jnp.float32)
    posq = jnp.concatenate([pos, batch_f[:, None], zeros4], axis=1)
    p8 = jnp.concatenate([pos, reflectance[:, None], zeros4], axis=1)
    sf8 = jnp.concatenate([sf, jnp.ones((4,), jnp.float32)]).reshape(1, 8)
    w1a = W1[:D_IN]
    w1b = jnp.concatenate([W1[D_IN:], jnp.zeros((4, H), jnp.float32)], axis=0)

    cid, hist = _cellhist(posq)
    p_pos, base = _rankpos(cid, hist)
    perm = p_pos.reshape(1, N)
    rows128 = jnp.concatenate([pos, batch_f[:, None], cid,
                               jnp.zeros((N, 123), jnp.float32)], axis=1)
    pos_srt = _sc_scatter(rows128, perm)
    pos_b = pos_srt[:, :8].reshape(N // 128, 128, 8).transpose(0, 2, 1)

    cid0 = pos_srt[::BQ, 4].astype(jnp.int32)
    cidl = pos_srt[BQ - 1::BQ, 4].astype(jnp.int32)
    starts = [
        jnp.clip(
            base[0, jnp.clip(cid0 + RUN_LO[r], 0, NC - 1)].astype(jnp.int32)
            // 128,
            0, N // 128 - RBLK)
        for r in range(NRUN)
    ]
    meta = jnp.stack(starts + [cid0, cidl]
                     + [jnp.zeros_like(cid0)] * 3, axis=0)

    idx, val = _search(meta, pos_srt, pos_b)

    amat, bmat = _precompute(x, p8, batch_f[:, None], sf8, w1a, w1b,
                             b1.reshape(1, H))
    a_s = _sc_scatter(amat, perm)
    b_s = _sc_scatter(bmat, perm)
    e = _sc_gather(a_s, idx.reshape(1, N * K))
    m, s, s2, cnt = _mlp(e, b_s, val, W2, b2.reshape(1, H))
    out_sorted = _finalize(m, s, s2, cnt, gamma.reshape(1, H),
                           beta.reshape(1, H))
    out = _sc_gather(out_sorted, perm)
    return (out, pos, batch, reflectance, sf)


# speedup vs baseline: 1.0421x; 1.0421x over previous
import functools

import jax
import jax.numpy as jnp
from jax.experimental import pallas as pl
from jax.experimental.pallas import tpu as pltpu
from jax.experimental.pallas import tpu_sc as plsc

N = 16384
K = 32
D_IN = 64
H = 128
R = 0.02 * 2.1
R2 = R * R
EPS = 1e-5
SLOPE = 0.01

G = 6
CS = R
NC = 1024
CHUNK = 512
BQG = 128
BQ = 256
RBLK = 6
NRUN = 3
RUN_LO = (-43, -7, 29)
RUN_HI = (-29, 7, 43)
W = NRUN * RBLK * 128
BN = 512
GATHER_WINDOW = 256

_HI = jax.lax.Precision.HIGHEST


def _leaky(v):
    return jnp.where(v >= 0, v, SLOPE * v)


def _cell_body(pq_ref, cid_ref, hist_ref):
    p = pq_ref[...]
    cx = jnp.clip(jnp.floor(p[:, 0:1] / CS), 0, G - 1)
    cy = jnp.clip(jnp.floor(p[:, 1:2] / CS), 0, G - 1)
    cz = jnp.clip(jnp.floor(p[:, 2:3] / CS), 0, G - 1)
    cid = ((p[:, 3:4] * G + cx) * G + cy) * G + cz
    cid_ref[...] = cid
    ci = jax.lax.broadcasted_iota(jnp.int32, (1, NC), 1).astype(jnp.float32)
    onehot = (cid == ci).astype(jnp.float32)

    @pl.when(pl.program_id(0) == 0)
    def _():
        hist_ref[...] = jnp.zeros_like(hist_ref)

    hist_ref[...] += jnp.sum(onehot, axis=0, keepdims=True)


def _cellhist(posq):
    return pl.pallas_call(
        _cell_body,
        grid=(N // CHUNK,),
        in_specs=[pl.BlockSpec((CHUNK, 8), lambda i: (i, 0))],
        out_specs=[
            pl.BlockSpec((CHUNK, 1), lambda i: (i, 0)),
            pl.BlockSpec((1, NC), lambda i: (0, 0)),
        ],
        out_shape=[
            jax.ShapeDtypeStruct((N, 1), jnp.float32),
            jax.ShapeDtypeStruct((1, NC), jnp.float32),
        ],
    )(posq)


def _rank_body(cid_ref, hist_ref, p_ref, base_ref, run_ref):
    ri = jax.lax.broadcasted_iota(jnp.int32, (NC, NC), 0)
    ci = jax.lax.broadcasted_iota(jnp.int32, (NC, NC), 1)

    @pl.when(pl.program_id(0) == 0)
    def _():
        run_ref[...] = jnp.zeros_like(run_ref)
        triu = (ri < ci).astype(jnp.float32)
        base_ref[...] = jax.lax.dot_general(
            hist_ref[...], triu, (((1,), (0,)), ((), ())),
            precision=_HI, preferred_element_type=jnp.float32)

    cid = cid_ref[...]
    lane = jax.lax.broadcasted_iota(jnp.int32, (1, NC), 1).astype(jnp.float32)
    onehot = (cid == lane).astype(jnp.float32)
    rj = jax.lax.broadcasted_iota(jnp.int32, (CHUNK, CHUNK), 0)
    cj = jax.lax.broadcasted_iota(jnp.int32, (CHUNK, CHUNK), 1)
    tril = (cj < rj).astype(jnp.float32)
    excl = jax.lax.dot_general(
        tril, onehot, (((1,), (0,)), ((), ())),
        precision=_HI, preferred_element_type=jnp.float32)
    dest = jnp.sum(onehot * (excl + run_ref[...] + base_ref[...]),
                   axis=1, keepdims=True)
    p_ref[...] = dest.astype(jnp.int32)
    run_ref[...] += jnp.sum(onehot, axis=0, keepdims=True)


def _rankpos(cid, hist):
    return pl.pallas_call(
        _rank_body,
        grid=(N // CHUNK,),
        in_specs=[
            pl.BlockSpec((CHUNK, 1), lambda i: (i, 0)),
            pl.BlockSpec((1, NC), lambda i: (0, 0)),
        ],
        out_specs=[
            pl.BlockSpec((CHUNK, 1), lambda i: (i, 0)),
            pl.BlockSpec((1, NC), lambda i: (0, 0)),
        ],
        out_shape=[
            jax.ShapeDtypeStruct((N, 1), jnp.int32),
            jax.ShapeDtypeStruct((1, NC), jnp.float32),
        ],
        scratch_shapes=[pltpu.VMEM((1, NC), jnp.float32)],
    )(cid, hist)


def _sc_scatter(rows, indices):
    num_idx = indices.shape[1]
    dim = rows.shape[1]
    mesh = plsc.VectorSubcoreMesh(core_axis_name="core",
                                  subcore_axis_name="subcore")

    @functools.partial(
        pl.kernel,
        out_type=jax.ShapeDtypeStruct((num_idx, dim), rows.dtype),
        mesh=mesh,
    )
    def scatter_kernel(x_hbm, i_hbm, o_hbm):
        def body(x_vmem, i_vmem):
            pltpu.sync_copy(x_vmem, o_hbm.at[i_vmem.at[0]])

        pltpu.emit_pipeline(
            body,
            grid=(num_idx // GATHER_WINDOW,),
            in_specs=[
                pl.BlockSpec((GATHER_WINDOW, dim), index_map=lambda i: (i, 0)),
                pl.BlockSpec((1, GATHER_WINDOW), index_map=lambda i: (0, i)),
            ],
            out_specs=[],
            core_axis_name=("core", "subcore"),
            dimension_semantics=(pltpu.PARALLEL,),
        )(x_hbm, i_hbm)

    return scatter_kernel(rows, indices)


def _sc_gather(table, indices):
    num_idx = indices.shape[1]
    dim = table.shape[1]
    mesh = plsc.VectorSubcoreMesh(core_axis_name="core",
                                  subcore_axis_name="subcore")

    @functools.partial(
        pl.kernel,
        out_type=jax.ShapeDtypeStruct((num_idx, dim), table.dtype),
        mesh=mesh,
    )
    def gather_kernel(x_hbm, i_hbm, o_hbm):
        def body(i_vmem, o_vmem):
            pltpu.sync_copy(x_hbm.at[i_vmem.at[0]], o_vmem)

        pltpu.emit_pipeline(
            body,
            grid=(num_idx // GATHER_WINDOW,),
            in_specs=[pl.BlockSpec((1, GATHER_WINDOW), index_map=lambda i: (0, i))],
            out_specs=[pl.BlockSpec((GATHER_WINDOW, dim), index_map=lambda i: (i, 0))],
            core_axis_name=("core", "subcore"),
            dimension_semantics=(pltpu.PARALLEL,),
        )(i_hbm, o_hbm)

    return gather_kernel(table, indices)


def _search_body(meta_ref, q_ref, pb_ref, idx_ref, val_ref):
    pid = pl.program_id(0)
    q = q_ref[...]
    inf = jnp.float32(jnp.inf)
    li0 = jax.lax.broadcasted_iota(jnp.int32, (BQG, 128), 1)
    kiota = jax.lax.broadcasted_iota(jnp.int32, (BQG, K), 1)

    states = []
    for h in range(BQ // BQG):
        col = (BQ // BQG) * pid + h
        qh = q[h * BQG:(h + 1) * BQG]
        qx = qh[:, 0:1]
        qy = qh[:, 1:2]
        qz = qh[:, 2:3]
        qb = qh[:, 3:4]
        cid0 = meta_ref[3, col]
        cidl = meta_ref[4, col]
        parts = []
        giotas = []
        for r in range(NRUN):
            sb = meta_ref[r, col]
            win = pb_ref[pl.ds(sb, RBLK)]
            lo = (cid0 + RUN_LO[r]).astype(jnp.float32)
            hi = (cidl + RUN_HI[r]).astype(jnp.float32)
            for c in range(RBLK):
                w = win[c]
                dx = qx - w[0:1, :]
                dy = qy - w[1:2, :]
                dz = qz - w[2:3, :]
                d2 = (dx * dx + dy * dy) + dz * dz
                wcid = w[4:5, :]
                m = ((qb == w[3:4, :]) & (d2 <= R2)
                     & (wcid >= lo) & (wcid <= hi))
                parts.append(jnp.where(m, d2, inf))
                giotas.append((sb + c) * 128 + li0)
        states.append((jnp.concatenate(parts, axis=1),
                       jnp.concatenate(giotas, axis=1),
                       jnp.zeros((BQG, K), jnp.int32),
                       jnp.zeros((BQG, K), jnp.float32)))

    def step(k, carry):
        out = []
        for d2m, giota, idxa, vala in carry:
            v = jnp.min(d2m, axis=1, keepdims=True)
            j = jnp.min(jnp.where(d2m == v, giota, N), axis=1, keepdims=True)
            idxa = jnp.where(kiota == k, j, idxa)
            vala = jnp.where(kiota == k, (v <= R2).astype(jnp.float32), vala)
            d2m = jnp.where(giota == j, inf, d2m)
            out.append((d2m, giota, idxa, vala))
        return tuple(out)

    final = jax.lax.fori_loop(0, K, step, tuple(states))
    idx_ref[...] = jnp.concatenate([s[2] for s in final], axis=0)
    val_ref[...] = jnp.concatenate([s[3] for s in final], axis=0)


def _search(meta, pos_srt, pos_b):
    return pl.pallas_call(
        _search_body,
        grid=(N // BQ,),
        in_specs=[
            pl.BlockSpec(memory_space=pltpu.SMEM),
            pl.BlockSpec((BQ, H), lambda i: (i, 0)),
            pl.BlockSpec((N // 128, 8, 128), lambda i: (0, 0, 0)),
        ],
        out_specs=[
            pl.BlockSpec((BQ, K), lambda i: (i, 0)),
            pl.BlockSpec((BQ, K), lambda i: (i, 0)),
        ],
        out_shape=[
            jax.ShapeDtypeStruct((N, K), jnp.int32),
            jax.ShapeDtypeStruct((N, K), jnp.float32),
        ],
    )(meta, pos_srt, pos_b)


def _pre_body(x_ref, p8_ref, bf_ref, sf_ref, w1a_ref, w1b_ref, b1_ref,
              a_ref, b_ref):
    p8 = p8_ref[...]
    bf = bf_ref[...]
    sfr = sf_ref[...]
    lane8 = jax.lax.broadcasted_iota(jnp.int32, (1, 8), 1)
    onehot = (bf == lane8.astype(jnp.float32)).astype(jnp.float32)
    s = jnp.sum(onehot * sfr, axis=1, keepdims=True)
    recip = 1.0 / s
    psc = jnp.where(lane8 < 3, p8 * recip, p8)
    bmat = jnp.dot(psc, w1b_ref[...], preferred_element_type=jnp.float32)
    amat = (jnp.dot(x_ref[...], w1a_ref[...],
                    preferred_element_type=jnp.float32) + bmat + b1_ref[...])
    a_ref[...] = amat
    b_ref[...] = bmat


def _precompute(x, p8, bf, sf8, w1a, w1b, b1r):
    blk = 2048
    return pl.pallas_call(
        _pre_body,
        grid=(N // blk,),
        in_specs=[
            pl.BlockSpec((blk, D_IN), lambda i: (i, 0)),
            pl.BlockSpec((blk, 8), lambda i: (i, 0)),
            pl.BlockSpec((blk, 1), lambda i: (i, 0)),
            pl.BlockSpec((1, 8), lambda i: (0, 0)),
            pl.BlockSpec((D_IN, H), lambda i: (0, 0)),
            pl.BlockSpec((8, H), lambda i: (0, 0)),
            pl.BlockSpec((1, H), lambda i: (0, 0)),
        ],
        out_specs=[
            pl.BlockSpec((blk, H), lambda i: (i, 0)),
            pl.BlockSpec((blk, H), lambda i: (i, 0)),
        ],
        out_shape=[
            jax.ShapeDtypeStruct((N, H), jnp.float32),
            jax.ShapeDtypeStruct((N, H), jnp.float32),
        ],
    )(x, p8, bf, sf8, w1a, w1b, b1r)


def _mlp_body(e_ref, b_ref, val_ref, w2_ref, b2_ref,
              m_ref, s_ref, s2_ref, cnt_ref):
    e = e_ref[...]
    bi = b_ref[...]
    val = val_ref[...]
    e3 = e.reshape(BN, K, H)
    h1 = _leaky(e3 - bi[:, None, :])
    h1f = h1.reshape(BN * K, H)
    h2 = _leaky(jnp.dot(h1f, w2_ref[...], preferred_element_type=jnp.float32)
                + b2_ref[...])
    h23 = h2.reshape(BN, K, H)
    v3 = val[:, :, None]
    m_ref[...] = jnp.max(jnp.where(v3 > 0, h23, -jnp.inf), axis=1)
    hv = h23 * v3
    s_part = jnp.sum(hv, axis=(0, 1), keepdims=False)
    s2_part = jnp.sum(hv * h23, axis=(0, 1), keepdims=False)
    cnt_part = jnp.sum(val)

    @pl.when(pl.program_id(0) == 0)
    def _():
        s_ref[...] = jnp.zeros_like(s_ref)
        s2_ref[...] = jnp.zeros_like(s2_ref)
        cnt_ref[...] = jnp.zeros_like(cnt_ref)

    s_ref[...] += s_part[None, :]
    s2_ref[...] += s2_part[None, :]
    cnt_ref[...] += jnp.full((1, H), cnt_part, dtype=jnp.float32)


def _mlp(e, bmat, val, w2, b2r):
    return pl.pallas_call(
        _mlp_body,
        grid=(N // BN,),
        in_specs=[
            pl.BlockSpec((BN * K, H), lambda i: (i, 0)),
            pl.BlockSpec((BN, H), lambda i: (i, 0)),
            pl.BlockSpec((BN, K), lambda i: (i, 0)),
            pl.BlockSpec((H, H), lambda i: (0, 0)),
            pl.BlockSpec((1, H), lambda i: (0, 0)),
        ],
        out_specs=[
            pl.BlockSpec((BN, H), lambda i: (i, 0)),
            pl.BlockSpec((1, H), lambda i: (0, 0)),
            pl.BlockSpec((1, H), lambda i: (0, 0)),
            pl.BlockSpec((1, H), lambda i: (0, 0)),
        ],
        out_shape=[
            jax.ShapeDtypeStruct((N, H), jnp.float32),
            jax.ShapeDtypeStruct((1, H), jnp.float32),
            jax.ShapeDtypeStruct((1, H), jnp.float32),
            jax.ShapeDtypeStruct((1, H), jnp.float32),
        ],
    )(e, bmat, val, w2, b2r)


def _fin_body(m_ref, s_ref, s2_ref, cnt_ref, g_ref, be_ref, o_ref):
    cnt = jnp.maximum(cnt_ref[0, 0], 1.0)
    mean = s_ref[...] / cnt
    var = jnp.maximum(s2_ref[...] / cnt - mean * mean, 0.0)
    inv = jax.lax.rsqrt(var + EPS) * g_ref[...]
    o_ref[...] = (m_ref[...] - mean) * inv + be_ref[...]


def _finalize(m, s, s2, cnt, gr, ber):
    blk = 2048
    return pl.pallas_call(
        _fin_body,
        grid=(N // blk,),
        in_specs=[
            pl.BlockSpec((blk, H), lambda i: (i, 0)),
            pl.BlockSpec((1, H), lambda i: (0, 0)),
            pl.BlockSpec((1, H), lambda i: (0, 0)),
            pl.BlockSpec((1, H), lambda i: (0, 0)),
            pl.BlockSpec((1, H), lambda i: (0, 0)),
            pl.BlockSpec((1, H), lambda i: (0, 0)),
        ],
        out_specs=pl.BlockSpec((blk, H), lambda i: (i, 0)),
        out_shape=jax.ShapeDtypeStruct((N, H), jnp.float32),
    )(m, s, s2, cnt, gr, ber)


def kernel(x, pos, batch, reflectance, sf, W1, b1, W2, b2, gamma, beta):
    batch_f = batch.astype(jnp.float32)
    zeros4 = jnp.zeros((N, 4), jnp.float32)
    posq = jnp.concatenate([pos, batch_f[:, None], zeros4], axis=1)
    p8 = jnp.concatenate([pos, reflectance[:, None], zeros4], axis=1)
    sf8 = jnp.concatenate([sf, jnp.ones((4,), jnp.float32)]).reshape(1, 8)
    w1a = W1[:D_IN]
    w1b = jnp.concatenate([W1[D_IN:], jnp.zeros((4, H), jnp.float32)], axis=0)

    cid, hist = _cellhist(posq)
    p_pos, base = _rankpos(cid, hist)
    perm = p_pos.reshape(1, N)
    rows128 = jnp.concatenate([pos, batch_f[:, None], cid,
                               jnp.zeros((N, 123), jnp.float32)], axis=1)
    pos_srt = _sc_scatter(rows128, perm)
    pos_b = pos_srt[:, :8].reshape(N // 128, 128, 8).transpose(0, 2, 1)

    cid0 = pos_srt[::BQG, 4].astype(jnp.int32)
    cidl = pos_srt[BQG - 1::BQG, 4].astype(jnp.int32)
    starts = [
        jnp.clip(
            base[0, jnp.clip(cid0 + RUN_LO[r], 0, NC - 1)].astype(jnp.int32)
            // 128,
            0, N // 128 - RBLK)
        for r in range(NRUN)
    ]
    meta = jnp.stack(starts + [cid0, cidl]
                     + [jnp.zeros_like(cid0)] * 3, axis=0)

    idx, val = _search(meta, pos_srt, pos_b)

    amat, bmat = _precompute(x, p8, batch_f[:, None], sf8, w1a, w1b,
                             b1.reshape(1, H))
    a_s = _sc_scatter(amat, perm)
    b_s = _sc_scatter(bmat, perm)
    e = _sc_gather(a_s, idx.reshape(1, N * K))
    m, s, s2, cnt = _mlp(e, b_s, val, W2, b2.reshape(1, H))
    out_sorted = _finalize(m, s, s2, cnt, gamma.reshape(1, H),
                           beta.reshape(1, H))
    out = _sc_gather(out_sorted, perm)
    return (out, pos, batch, reflectance, sf)


# speedup vs baseline: 1.0897x; 1.0456x over previous
import functools

import jax
import jax.numpy as jnp
from jax.experimental import pallas as pl
from jax.experimental.pallas import tpu as pltpu
from jax.experimental.pallas import tpu_sc as plsc

N = 16384
K = 32
D_IN = 64
H = 128
R = 0.02 * 2.1
R2 = R * R
EPS = 1e-5
SLOPE = 0.01

G = 6
CS = R
NC = 1024
CHUNK = 512
BQG = 128
BQ = 512
RBLK = 6
NRUN = 3
RUN_LO = (-43, -7, 29)
RUN_HI = (-29, 7, 43)
W = NRUN * RBLK * 128
BN = 512
GATHER_WINDOW = 256

_HI = jax.lax.Precision.HIGHEST


def _leaky(v):
    return jnp.where(v >= 0, v, SLOPE * v)


def _cell_body(pq_ref, cid_ref, hist_ref):
    p = pq_ref[...]
    cx = jnp.clip(jnp.floor(p[:, 0:1] / CS), 0, G - 1)
    cy = jnp.clip(jnp.floor(p[:, 1:2] / CS), 0, G - 1)
    cz = jnp.clip(jnp.floor(p[:, 2:3] / CS), 0, G - 1)
    cid = ((p[:, 3:4] * G + cx) * G + cy) * G + cz
    cid_ref[...] = cid
    ci = jax.lax.broadcasted_iota(jnp.int32, (1, NC), 1).astype(jnp.float32)
    onehot = (cid == ci).astype(jnp.float32)

    @pl.when(pl.program_id(0) == 0)
    def _():
        hist_ref[...] = jnp.zeros_like(hist_ref)

    hist_ref[...] += jnp.sum(onehot, axis=0, keepdims=True)


def _cellhist(posq):
    return pl.pallas_call(
        _cell_body,
        grid=(N // CHUNK,),
        in_specs=[pl.BlockSpec((CHUNK, 8), lambda i: (i, 0))],
        out_specs=[
            pl.BlockSpec((CHUNK, 1), lambda i: (i, 0)),
            pl.BlockSpec((1, NC), lambda i: (0, 0)),
        ],
        out_shape=[
            jax.ShapeDtypeStruct((N, 1), jnp.float32),
            jax.ShapeDtypeStruct((1, NC), jnp.float32),
        ],
    )(posq)


def _rank_body(cid_ref, hist_ref, p_ref, base_ref, run_ref):
    ri = jax.lax.broadcasted_iota(jnp.int32, (NC, NC), 0)
    ci = jax.lax.broadcasted_iota(jnp.int32, (NC, NC), 1)

    @pl.when(pl.program_id(0) == 0)
    def _():
        run_ref[...] = jnp.zeros_like(run_ref)
        triu = (ri < ci).astype(jnp.float32)
        base_ref[...] = jax.lax.dot_general(
            hist_ref[...], triu, (((1,), (0,)), ((), ())),
            precision=_HI, preferred_element_type=jnp.float32)

    cid = cid_ref[...]
    lane = jax.lax.broadcasted_iota(jnp.int32, (1, NC), 1).astype(jnp.float32)
    onehot = (cid == lane).astype(jnp.float32)
    rj = jax.lax.broadcasted_iota(jnp.int32, (CHUNK, CHUNK), 0)
    cj = jax.lax.broadcasted_iota(jnp.int32, (CHUNK, CHUNK), 1)
    tril = (cj < rj).astype(jnp.float32)
    excl = jax.lax.dot_general(
        tril, onehot, (((1,), (0,)), ((), ())),
        precision=_HI, preferred_element_type=jnp.float32)
    dest = jnp.sum(onehot * (excl + run_ref[...] + base_ref[...]),
                   axis=1, keepdims=True)
    p_ref[...] = dest.astype(jnp.int32)
    run_ref[...] += jnp.sum(onehot, axis=0, keepdims=True)


def _rankpos(cid, hist):
    return pl.pallas_call(
        _rank_body,
        grid=(N // CHUNK,),
        in_specs=[
            pl.BlockSpec((CHUNK, 1), lambda i: (i, 0)),
            pl.BlockSpec((1, NC), lambda i: (0, 0)),
        ],
        out_specs=[
            pl.BlockSpec((CHUNK, 1), lambda i: (i, 0)),
            pl.BlockSpec((1, NC), lambda i: (0, 0)),
        ],
        out_shape=[
            jax.ShapeDtypeStruct((N, 1), jnp.int32),
            jax.ShapeDtypeStruct((1, NC), jnp.float32),
        ],
        scratch_shapes=[pltpu.VMEM((1, NC), jnp.float32)],
    )(cid, hist)


def _sc_scatter(rows, indices):
    num_idx = indices.shape[1]
    dim = rows.shape[1]
    mesh = plsc.VectorSubcoreMesh(core_axis_name="core",
                                  subcore_axis_name="subcore")

    @functools.partial(
        pl.kernel,
        out_type=jax.ShapeDtypeStruct((num_idx, dim), rows.dtype),
        mesh=mesh,
    )
    def scatter_kernel(x_hbm, i_hbm, o_hbm):
        def body(x_vmem, i_vmem):
            pltpu.sync_copy(x_vmem, o_hbm.at[i_vmem.at[0]])

        pltpu.emit_pipeline(
            body,
            grid=(num_idx // GATHER_WINDOW,),
            in_specs=[
                pl.BlockSpec((GATHER_WINDOW, dim), index_map=lambda i: (i, 0)),
                pl.BlockSpec((1, GATHER_WINDOW), index_map=lambda i: (0, i)),
            ],
            out_specs=[],
            core_axis_name=("core", "subcore"),
            dimension_semantics=(pltpu.PARALLEL,),
        )(x_hbm, i_hbm)

    return scatter_kernel(rows, indices)


def _sc_gather(table, indices):
    num_idx = indices.shape[1]
    dim = table.shape[1]
    mesh = plsc.VectorSubcoreMesh(core_axis_name="core",
                                  subcore_axis_name="subcore")

    @functools.partial(
        pl.kernel,
        out_type=jax.ShapeDtypeStruct((num_idx, dim), table.dtype),
        mesh=mesh,
    )
    def gather_kernel(x_hbm, i_hbm, o_hbm):
        def body(i_vmem, o_vmem):
            pltpu.sync_copy(x_hbm.at[i_vmem.at[0]], o_vmem)

        pltpu.emit_pipeline(
            body,
            grid=(num_idx // GATHER_WINDOW,),
            in_specs=[pl.BlockSpec((1, GATHER_WINDOW), index_map=lambda i: (0, i))],
            out_specs=[pl.BlockSpec((GATHER_WINDOW, dim), index_map=lambda i: (i, 0))],
            core_axis_name=("core", "subcore"),
            dimension_semantics=(pltpu.PARALLEL,),
        )(i_hbm, o_hbm)

    return gather_kernel(table, indices)


def _search_body(meta_ref, q_ref, pb_ref, idx_ref, val_ref):
    pid = pl.program_id(0)
    q = q_ref[...]
    inf = jnp.float32(jnp.inf)
    li0 = jax.lax.broadcasted_iota(jnp.int32, (BQG, 128), 1)
    kiota = jax.lax.broadcasted_iota(jnp.int32, (BQG, K), 1)

    states = []
    for h in range(BQ // BQG):
        col = (BQ // BQG) * pid + h
        qh = q[h * BQG:(h + 1) * BQG]
        qx = qh[:, 0:1]
        qy = qh[:, 1:2]
        qz = qh[:, 2:3]
        qb = qh[:, 3:4]
        cid0 = meta_ref[3, col]
        cidl = meta_ref[4, col]
        parts = []
        giotas = []
        for r in range(NRUN):
            sb = meta_ref[r, col]
            win = pb_ref[pl.ds(sb, RBLK)]
            lo = (cid0 + RUN_LO[r]).astype(jnp.float32)
            hi = (cidl + RUN_HI[r]).astype(jnp.float32)
            for c in range(RBLK):
                w = win[c]
                dx = qx - w[0:1, :]
                dy = qy - w[1:2, :]
                dz = qz - w[2:3, :]
                d2 = (dx * dx + dy * dy) + dz * dz
                wcid = w[4:5, :]
                m = ((qb == w[3:4, :]) & (d2 <= R2)
                     & (wcid >= lo) & (wcid <= hi))
                parts.append(jnp.where(m, d2, inf))
                giotas.append((sb + c) * 128 + li0)
        states.append((jnp.concatenate(parts, axis=1),
                       jnp.concatenate(giotas, axis=1),
                       jnp.zeros((BQG, K), jnp.int32),
                       jnp.zeros((BQG, K), jnp.float32)))

    def step(k, carry):
        out = []
        for d2m, giota, idxa, vala in carry:
            v = jnp.min(d2m, axis=1, keepdims=True)
            j = jnp.min(jnp.where(d2m == v, giota, N), axis=1, keepdims=True)
            idxa = jnp.where(kiota == k, j, idxa)
            vala = jnp.where(kiota == k, (v <= R2).astype(jnp.float32), vala)
            d2m = jnp.where(giota == j, inf, d2m)
            out.append((d2m, giota, idxa, vala))
        return tuple(out)

    final = jax.lax.fori_loop(0, K, step, tuple(states))
    idx_ref[...] = jnp.concatenate([s[2] for s in final], axis=0)
    val_ref[...] = jnp.concatenate([s[3] for s in final], axis=0)


def _search(meta, pos_srt, pos_b):
    return pl.pallas_call(
        _search_body,
        grid=(N // BQ,),
        in_specs=[
            pl.BlockSpec(memory_space=pltpu.SMEM),
            pl.BlockSpec((BQ, H), lambda i: (i, 0)),
            pl.BlockSpec((N // 128, 8, 128), lambda i: (0, 0, 0)),
        ],
        out_specs=[
            pl.BlockSpec((BQ, K), lambda i: (i, 0)),
            pl.BlockSpec((BQ, K), lambda i: (i, 0)),
        ],
        out_shape=[
            jax.ShapeDtypeStruct((N, K), jnp.int32),
            jax.ShapeDtypeStruct((N, K), jnp.float32),
        ],
    )(meta, pos_srt, pos_b)


def _pre_body(x_ref, p8_ref, bf_ref, sf_ref, w1a_ref, w1b_ref, b1_ref,
              a_ref, b_ref):
    p8 = p8_ref[...]
    bf = bf_ref[...]
    sfr = sf_ref[...]
    lane8 = jax.lax.broadcasted_iota(jnp.int32, (1, 8), 1)
    onehot = (bf == lane8.astype(jnp.float32)).astype(jnp.float32)
    s = jnp.sum(onehot * sfr, axis=1, keepdims=True)
    recip = 1.0 / s
    psc = jnp.where(lane8 < 3, p8 * recip, p8)
    bmat = jnp.dot(psc, w1b_ref[...], preferred_element_type=jnp.float32)
    amat = (jnp.dot(x_ref[...], w1a_ref[...],
                    preferred_element_type=jnp.float32) + bmat + b1_ref[...])
    a_ref[...] = amat
    b_ref[...] = bmat


def _precompute(x, p8, bf, sf8, w1a, w1b, b1r):
    blk = 2048
    return pl.pallas_call(
        _pre_body,
        grid=(N // blk,),
        in_specs=[
            pl.BlockSpec((blk, D_IN), lambda i: (i, 0)),
            pl.BlockSpec((blk, 8), lambda i: (i, 0)),
            pl.BlockSpec((blk, 1), lambda i: (i, 0)),
            pl.BlockSpec((1, 8), lambda i: (0, 0)),
            pl.BlockSpec((D_IN, H), lambda i: (0, 0)),
            pl.BlockSpec((8, H), lambda i: (0, 0)),
            pl.BlockSpec((1, H), lambda i: (0, 0)),
        ],
        out_specs=[
            pl.BlockSpec((blk, H), lambda i: (i, 0)),
            pl.BlockSpec((blk, H), lambda i: (i, 0)),
        ],
        out_shape=[
            jax.ShapeDtypeStruct((N, H), jnp.float32),
            jax.ShapeDtypeStruct((N, H), jnp.float32),
        ],
    )(x, p8, bf, sf8, w1a, w1b, b1r)


def _mlp_body(e_ref, b_ref, val_ref, w2_ref, b2_ref,
              m_ref, s_ref, s2_ref, cnt_ref):
    e = e_ref[...]
    bi = b_ref[...]
    val = val_ref[...]
    e3 = e.reshape(BN, K, H)
    h1 = _leaky(e3 - bi[:, None, :])
    h1f = h1.reshape(BN * K, H)
    h2 = _leaky(jnp.dot(h1f, w2_ref[...], preferred_element_type=jnp.float32)
                + b2_ref[...])
    h23 = h2.reshape(BN, K, H)
    v3 = val[:, :, None]
    m_ref[...] = jnp.max(jnp.where(v3 > 0, h23, -jnp.inf), axis=1)
    hv = h23 * v3
    s_part = jnp.sum(hv, axis=(0, 1), keepdims=False)
    s2_part = jnp.sum(hv * h23, axis=(0, 1), keepdims=False)
    cnt_part = jnp.sum(val)

    @pl.when(pl.program_id(0) == 0)
    def _():
        s_ref[...] = jnp.zeros_like(s_ref)
        s2_ref[...] = jnp.zeros_like(s2_ref)
        cnt_ref[...] = jnp.zeros_like(cnt_ref)

    s_ref[...] += s_part[None, :]
    s2_ref[...] += s2_part[None, :]
    cnt_ref[...] += jnp.full((1, H), cnt_part, dtype=jnp.float32)


def _mlp(e, bmat, val, w2, b2r):
    return pl.pallas_call(
        _mlp_body,
        grid=(N // BN,),
        in_specs=[
            pl.BlockSpec((BN * K, H), lambda i: (i, 0)),
            pl.BlockSpec((BN, H), lambda i: (i, 0)),
            pl.BlockSpec((BN, K), lambda i: (i, 0)),
            pl.BlockSpec((H, H), lambda i: (0, 0)),
            pl.BlockSpec((1, H), lambda i: (0, 0)),
        ],
        out_specs=[
            pl.BlockSpec((BN, H), lambda i: (i, 0)),
            pl.BlockSpec((1, H), lambda i: (0, 0)),
            pl.BlockSpec((1, H), lambda i: (0, 0)),
            pl.BlockSpec((1, H), lambda i: (0, 0)),
        ],
        out_shape=[
            jax.ShapeDtypeStruct((N, H), jnp.float32),
            jax.ShapeDtypeStruct((1, H), jnp.float32),
            jax.ShapeDtypeStruct((1, H), jnp.float32),
            jax.ShapeDtypeStruct((1, H), jnp.float32),
        ],
    )(e, bmat, val, w2, b2r)


def _fin_body(m_ref, s_ref, s2_ref, cnt_ref, g_ref, be_ref, o_ref):
    cnt = jnp.maximum(cnt_ref[0, 0], 1.0)
    mean = s_ref[...] / cnt
    var = jnp.maximum(s2_ref[...] / cnt - mean * mean, 0.0)
    inv = jax.lax.rsqrt(var + EPS) * g_ref[...]
    o_ref[...] = (m_ref[...] - mean) * inv + be_ref[...]


def _finalize(m, s, s2, cnt, gr, ber):
    blk = 2048
    return pl.pallas_call(
        _fin_body,
        grid=(N // blk,),
        in_specs=[
            pl.BlockSpec((blk, H), lambda i: (i, 0)),
            pl.BlockSpec((1, H), lambda i: (0, 0)),
            pl.BlockSpec((1, H), lambda i: (0, 0)),
            pl.BlockSpec((1, H), lambda i: (0, 0)),
            pl.BlockSpec((1, H), lambda i: (0, 0)),
            pl.BlockSpec((1, H), lambda i: (0, 0)),
        ],
        out_specs=pl.BlockSpec((blk, H), lambda i: (i, 0)),
        out_shape=jax.ShapeDtypeStruct((N, H), jnp.float32),
    )(m, s, s2, cnt, gr, ber)


def kernel(x, pos, batch, reflectance, sf, W1, b1, W2, b2, gamma, beta):
    batch_f = batch.astype(jnp.float32)
    zeros4 = jnp.zeros((N, 4), jnp.float32)
    posq = jnp.concatenate([pos, batch_f[:, None], zeros4], axis=1)
    p8 = jnp.concatenate([pos, reflectance[:, None], zeros4], axis=1)
    sf8 = jnp.concatenate([sf, jnp.ones((4,), jnp.float32)]).reshape(1, 8)
    w1a = W1[:D_IN]
    w1b = jnp.concatenate([W1[D_IN:], jnp.zeros((4, H), jnp.float32)], axis=0)

    cid, hist = _cellhist(posq)
    p_pos, base = _rankpos(cid, hist)
    perm = p_pos.reshape(1, N)
    rows128 = jnp.concatenate([pos, batch_f[:, None], cid,
                               jnp.zeros((N, 123), jnp.float32)], axis=1)
    pos_srt = _sc_scatter(rows128, perm)
    pos_b = pos_srt[:, :8].reshape(N // 128, 128, 8).transpose(0, 2, 1)

    cid0 = pos_srt[::BQG, 4].astype(jnp.int32)
    cidl = pos_srt[BQG - 1::BQG, 4].astype(jnp.int32)
    starts = [
        jnp.clip(
            base[0, jnp.clip(cid0 + RUN_LO[r], 0, NC - 1)].astype(jnp.int32)
            // 128,
            0, N // 128 - RBLK)
        for r in range(NRUN)
    ]
    meta = jnp.stack(starts + [cid0, cidl]
                     + [jnp.zeros_like(cid0)] * 3, axis=0)

    idx, val = _search(meta, pos_srt, pos_b)

    amat, bmat = _precompute(x, p8, batch_f[:, None], sf8, w1a, w1b,
                             b1.reshape(1, H))
    a_s = _sc_scatter(amat, perm)
    b_s = _sc_scatter(bmat, perm)
    e = _sc_gather(a_s, idx.reshape(1, N * K))
    m, s, s2, cnt = _mlp(e, b_s, val, W2, b2.reshape(1, H))
    out_sorted = _finalize(m, s, s2, cnt, gamma.reshape(1, H),
                           beta.reshape(1, H))
    out = _sc_gather(out_sorted, perm)
    return (out, pos, batch, reflectance, sf)
